# Initial kernel scaffold; baseline (speedup 1.0000x reference)
import jax
import jax.numpy as jnp
from jax.experimental import pallas as pl


def kernel(tokens, table):
    raise NotImplementedError("write your pallas kernel here")


# speedup vs baseline: 1.5828x; 1.5828x over previous
import functools

import jax
import jax.numpy as jnp
from jax import lax
from jax.experimental import pallas as pl
from jax.experimental.pallas import tpu as pltpu
from jax.experimental.pallas import tpu_sc as plsc

_NC = 2
_NS = 16
_NW = _NC * _NS

_C = 64
_NB = 2


@functools.lru_cache(maxsize=None)
def _build(n_tokens, vocab, dim):
    assert n_tokens % (_NW * _C) == 0
    b_per_w = n_tokens // _NW
    n_chunks = b_per_w // _C

    mesh = plsc.VectorSubcoreMesh(core_axis_name="c", subcore_axis_name="s")

    @functools.partial(
        pl.kernel,
        mesh=mesh,
        out_type=jax.ShapeDtypeStruct((n_tokens, dim), jnp.float32),
        scratch_types=[
            pltpu.VMEM((b_per_w,), jnp.int32),
            pltpu.VMEM((_NB * _C, dim), jnp.float32),
            pltpu.SemaphoreType.DMA,
            pltpu.SemaphoreType.DMA,
            pltpu.SemaphoreType.DMA,
            pltpu.SemaphoreType.DMA,
        ],
    )
    def gather_kernel(table_hbm, idx_hbm, out_hbm, idx_v, bufs, g0, g1, o0, o1):
        wid = lax.axis_index("s") * _NC + lax.axis_index("c")
        base = wid * b_per_w
        gsems = (g0, g1)
        osems = (o0, o1)

        pltpu.sync_copy(idx_hbm.at[pl.ds(base, b_per_w)], idx_v)

        ghandles = [None] * n_chunks
        ohandles = [None] * n_chunks

        def start_gather(c):
            b = c % _NB
            ghandles[c] = pltpu.async_copy(
                table_hbm.at[idx_v.at[pl.ds(c * _C, _C)]],
                bufs.at[pl.ds(b * _C, _C)],
                gsems[b],
            )

        for c in range(_NB):
            start_gather(c)

        for c in range(n_chunks):
            b = c % _NB
            ghandles[c].wait()
            ohandles[c] = pltpu.async_copy(
                bufs.at[pl.ds(b * _C, _C)],
                out_hbm.at[pl.ds(base + c * _C, _C)],
                osems[b],
            )
            nc = c + _NB
            if nc < n_chunks:
                ohandles[c].wait()
                start_gather(nc)

        for c in range(n_chunks - _NB, n_chunks):
            if ohandles[c] is not None:
                ohandles[c].wait()

    return gather_kernel


def kernel(tokens, table):
    bsz, seq = tokens.shape
    vocab, dim = table.shape
    idx = tokens.reshape(-1)
    out = _build(bsz * seq, vocab, dim)(table, idx)
    return out.reshape(bsz, seq, dim)


# speedup vs baseline: 1.5913x; 1.0054x over previous
import functools

import jax
import jax.numpy as jnp
from jax import lax
from jax.experimental import pallas as pl
from jax.experimental.pallas import tpu as pltpu
from jax.experimental.pallas import tpu_sc as plsc

_NC = 2
_NS = 16
_NW = _NC * _NS

_C = 32
_NB = 4


@functools.lru_cache(maxsize=None)
def _build(n_tokens, vocab, dim):
    assert n_tokens % (_NW * _C) == 0
    b_per_w = n_tokens // _NW
    n_chunks = b_per_w // _C

    mesh = plsc.VectorSubcoreMesh(core_axis_name="c", subcore_axis_name="s")

    @functools.partial(
        pl.kernel,
        mesh=mesh,
        out_type=jax.ShapeDtypeStruct((n_tokens, dim), jnp.float32),
        scratch_types=[
            pltpu.VMEM((b_per_w,), jnp.int32),
            pltpu.VMEM((_NB * _C, dim), jnp.float32),
        ] + [pltpu.SemaphoreType.DMA] * (2 * _NB),
    )
    def gather_kernel(table_hbm, idx_hbm, out_hbm, idx_v, bufs, *sems):
        wid = lax.axis_index("s") * _NC + lax.axis_index("c")
        base = wid * b_per_w
        gsems = sems[:_NB]
        osems = sems[_NB:]

        pltpu.sync_copy(idx_hbm.at[pl.ds(base, b_per_w)], idx_v)

        ghandles = [None] * n_chunks
        ohandles = [None] * n_chunks

        def start_gather(c):
            b = c % _NB
            ghandles[c] = pltpu.async_copy(
                table_hbm.at[idx_v.at[pl.ds(c * _C, _C)]],
                bufs.at[pl.ds(b * _C, _C)],
                gsems[b],
            )

        for c in range(_NB):
            start_gather(c)

        for c in range(n_chunks):
            b = c % _NB
            ghandles[c].wait()
            ohandles[c] = pltpu.async_copy(
                bufs.at[pl.ds(b * _C, _C)],
                out_hbm.at[pl.ds(base + c * _C, _C)],
                osems[b],
            )
            nc = c + _NB
            if nc < n_chunks:
                ohandles[c].wait()
                start_gather(nc)

        for c in range(n_chunks - _NB, n_chunks):
            if ohandles[c] is not None:
                ohandles[c].wait()

    return gather_kernel


def kernel(tokens, table):
    bsz, seq = tokens.shape
    vocab, dim = table.shape
    idx = tokens.reshape(-1)
    out = _build(bsz * seq, vocab, dim)(table, idx)
    return out.reshape(bsz, seq, dim)


# speedup vs baseline: 1.5943x; 1.0019x over previous
import functools

import jax
import jax.numpy as jnp
from jax import lax
from jax.experimental import pallas as pl
from jax.experimental.pallas import tpu as pltpu
from jax.experimental.pallas import tpu_sc as plsc

_NC = 2
_NS = 16
_NW = _NC * _NS

_C = 32
_NB = 4


@functools.lru_cache(maxsize=None)
def _build(bsz, seq, vocab, dim):
    assert bsz == _NW and seq % _C == 0
    n_chunks = seq // _C

    mesh = plsc.VectorSubcoreMesh(core_axis_name="c", subcore_axis_name="s")

    @functools.partial(
        pl.kernel,
        mesh=mesh,
        out_type=jax.ShapeDtypeStruct((bsz, seq, dim), jnp.float32),
        scratch_types=[
            pltpu.VMEM((seq,), jnp.int32),
            pltpu.VMEM((_NB * _C, dim), jnp.float32),
        ] + [pltpu.SemaphoreType.DMA] * (2 * _NB),
    )
    def gather_kernel(table_hbm, tok_hbm, out_hbm, idx_v, bufs, *sems):
        wid = lax.axis_index("s") * _NC + lax.axis_index("c")
        gsems = sems[:_NB]
        osems = sems[_NB:]

        pltpu.sync_copy(tok_hbm.at[wid], idx_v)

        ghandles = [None] * n_chunks
        ohandles = [None] * n_chunks

        def start_gather(c):
            b = c % _NB
            ghandles[c] = pltpu.async_copy(
                table_hbm.at[idx_v.at[pl.ds(c * _C, _C)]],
                bufs.at[pl.ds(b * _C, _C)],
                gsems[b],
            )

        for c in range(_NB):
            start_gather(c)

        for c in range(n_chunks):
            b = c % _NB
            ghandles[c].wait()
            ohandles[c] = pltpu.async_copy(
                bufs.at[pl.ds(b * _C, _C)],
                out_hbm.at[wid, pl.ds(c * _C, _C)],
                osems[b],
            )
            nc = c + _NB
            if nc < n_chunks:
                ohandles[c].wait()
                start_gather(nc)

        for c in range(n_chunks - _NB, n_chunks):
            if ohandles[c] is not None:
                ohandles[c].wait()

    return gather_kernel


def kernel(tokens, table):
    bsz, seq = tokens.shape
    vocab, dim = table.shape
    return _build(bsz, seq, vocab, dim)(table, tokens)
